# baseline (speedup 1.0000x reference)
import jax
import jax.numpy as jnp
from jax.experimental import pallas as pl
from jax.experimental.pallas import tpu as pltpu

_N = 10000
_BR = 400
_NB = _N // _BR
_DOT_PREC = jax.lax.Precision.HIGHEST


def _net_kernel(xt_ref, g_ref, a_ref, wbd_ref, bias_ref, w4_ref, b4_ref,
                out_ref, z_ref, h_ref, acc_ref):
    l = pl.program_id(0)
    i = pl.program_id(1)

    @pl.when(i == 0)
    def _start_layer():
        @pl.when(l == 0)
        def _():
            z_ref[...] = (
                jnp.dot(xt_ref[...], g_ref[...],
                        preferred_element_type=jnp.float32)
                + bias_ref[0]
            )

        @pl.when(l > 0)
        def _():
            z_ref[...] = (
                jnp.dot(h_ref[...], wbd_ref[0],
                        preferred_element_type=jnp.float32)
                + bias_ref[0]
            )

        acc_ref[...] = jnp.zeros_like(acc_ref)

    h_blk = jnp.maximum(
        jnp.dot(a_ref[...], z_ref[...],
                preferred_element_type=jnp.float32, precision=_DOT_PREC),
        0.0,
    )
    h_ref[pl.ds(i * _BR, _BR), :] = h_blk

    @pl.when(l == 2)
    def _tail():
        acc_ref[...] += jnp.sum(h_blk, axis=0, keepdims=True)

        @pl.when(i == _NB - 1)
        def _head():
            m = acc_ref[...] * (1.0 / _N)
            out_ref[...] = (
                jnp.dot(m, w4_ref[...], preferred_element_type=jnp.float32)
                + b4_ref[...]
            )


def kernel(x, A, W1, b1, W2, b2, W3, b3, W4, b4):
    B, N, _ = x.shape
    H = W2.shape[0]
    OUT = W4.shape[1]
    D = B * H

    f32 = jnp.float32
    xt = x[:, :, 0].T.astype(f32)

    g = jnp.zeros((B, D), f32)
    g = g.at[0, :H].set(W1[:, :].reshape(H))
    g = g.at[1, H:].set(W1[:, :].reshape(H))

    def blockdiag(W):
        Z = jnp.zeros((D, D), f32)
        return Z.at[:H, :H].set(W).at[H:, H:].set(W)

    wbd = jnp.stack([blockdiag(W2), blockdiag(W3)])
    biases = jnp.stack([jnp.tile(b1, B), jnp.tile(b2, B),
                        jnp.tile(b3, B)])[:, None, :]

    w4bd = jnp.zeros((D, B * OUT), f32)
    w4bd = w4bd.at[:H, :OUT].set(W4).at[H:, OUT:].set(W4)
    b4t = jnp.tile(b4, B)[None, :]

    out = pl.pallas_call(
        _net_kernel,
        grid=(3, _NB),
        in_specs=[
            pl.BlockSpec((N, B), lambda l, i: (0, 0)),
            pl.BlockSpec((B, D), lambda l, i: (0, 0)),
            pl.BlockSpec((_BR, N), lambda l, i: (i, 0)),
            pl.BlockSpec((1, D, D),
                         lambda l, i: (jnp.maximum(l - 1, 0), 0, 0)),
            pl.BlockSpec((1, 1, D), lambda l, i: (l, 0, 0)),
            pl.BlockSpec((D, B * OUT), lambda l, i: (0, 0)),
            pl.BlockSpec((1, B * OUT), lambda l, i: (0, 0)),
        ],
        out_specs=pl.BlockSpec((1, B * OUT), lambda l, i: (0, 0)),
        out_shape=jax.ShapeDtypeStruct((1, B * OUT), f32),
        scratch_shapes=[
            pltpu.VMEM((N, D), f32),
            pltpu.VMEM((N, D), f32),
            pltpu.VMEM((1, D), f32),
        ],
        compiler_params=pltpu.CompilerParams(
            dimension_semantics=("arbitrary", "arbitrary"),
        ),
    )(xt, g, A, wbd, biases, w4bd, b4t)

    return out.reshape(B, OUT)


# speedup vs baseline: 2.5853x; 2.5853x over previous
import jax
import jax.numpy as jnp
from jax.experimental import pallas as pl
from jax.experimental.pallas import tpu as pltpu

_N = 10000
_BR = 400
_NB = _N // _BR
_DOT_PREC = jax.lax.Precision.DEFAULT


def _net_kernel(xt_ref, g_ref, a_ref, wbd_ref, bias_ref, w4_ref, b4_ref,
                out_ref, z_ref, h_ref, acc_ref):
    l = pl.program_id(0)
    i = pl.program_id(1)

    @pl.when(i == 0)
    def _start_layer():
        @pl.when(l == 0)
        def _():
            z_ref[...] = (
                jnp.dot(xt_ref[...], g_ref[...],
                        preferred_element_type=jnp.float32)
                + bias_ref[0]
            )

        @pl.when(l > 0)
        def _():
            z_ref[...] = (
                jnp.dot(h_ref[...], wbd_ref[0],
                        preferred_element_type=jnp.float32)
                + bias_ref[0]
            )

        acc_ref[...] = jnp.zeros_like(acc_ref)

    h_blk = jnp.maximum(
        jnp.dot(a_ref[...], z_ref[...],
                preferred_element_type=jnp.float32, precision=_DOT_PREC),
        0.0,
    )
    h_ref[pl.ds(i * _BR, _BR), :] = h_blk

    @pl.when(l == 2)
    def _tail():
        acc_ref[...] += jnp.sum(h_blk, axis=0, keepdims=True)

        @pl.when(i == _NB - 1)
        def _head():
            m = acc_ref[...] * (1.0 / _N)
            out_ref[...] = (
                jnp.dot(m, w4_ref[...], preferred_element_type=jnp.float32)
                + b4_ref[...]
            )


def kernel(x, A, W1, b1, W2, b2, W3, b3, W4, b4):
    B, N, _ = x.shape
    H = W2.shape[0]
    OUT = W4.shape[1]
    D = B * H

    f32 = jnp.float32
    xt = x[:, :, 0].T.astype(f32)

    g = jnp.zeros((B, D), f32)
    g = g.at[0, :H].set(W1[:, :].reshape(H))
    g = g.at[1, H:].set(W1[:, :].reshape(H))

    def blockdiag(W):
        Z = jnp.zeros((D, D), f32)
        return Z.at[:H, :H].set(W).at[H:, H:].set(W)

    wbd = jnp.stack([blockdiag(W2), blockdiag(W3)])
    biases = jnp.stack([jnp.tile(b1, B), jnp.tile(b2, B),
                        jnp.tile(b3, B)])[:, None, :]

    w4bd = jnp.zeros((D, B * OUT), f32)
    w4bd = w4bd.at[:H, :OUT].set(W4).at[H:, OUT:].set(W4)
    b4t = jnp.tile(b4, B)[None, :]

    out = pl.pallas_call(
        _net_kernel,
        grid=(3, _NB),
        in_specs=[
            pl.BlockSpec((N, B), lambda l, i: (0, 0)),
            pl.BlockSpec((B, D), lambda l, i: (0, 0)),
            pl.BlockSpec((_BR, N), lambda l, i: (i, 0)),
            pl.BlockSpec((1, D, D),
                         lambda l, i: (jnp.maximum(l - 1, 0), 0, 0)),
            pl.BlockSpec((1, 1, D), lambda l, i: (l, 0, 0)),
            pl.BlockSpec((D, B * OUT), lambda l, i: (0, 0)),
            pl.BlockSpec((1, B * OUT), lambda l, i: (0, 0)),
        ],
        out_specs=pl.BlockSpec((1, B * OUT), lambda l, i: (0, 0)),
        out_shape=jax.ShapeDtypeStruct((1, B * OUT), f32),
        scratch_shapes=[
            pltpu.VMEM((N, D), f32),
            pltpu.VMEM((N, D), f32),
            pltpu.VMEM((1, D), f32),
        ],
        compiler_params=pltpu.CompilerParams(
            dimension_semantics=("arbitrary", "arbitrary"),
        ),
    )(xt, g, A, wbd, biases, w4bd, b4t)

    return out.reshape(B, OUT)
